# baseline (speedup 1.0000x reference)
import functools

import jax
import jax.numpy as jnp
from jax import lax
from jax.experimental import pallas as pl
from jax.experimental.pallas import tpu as pltpu
from jax.experimental.pallas import tpu_sc as plsc

POSE_NUM = 100000
EMBED_DIM = 6
BATCH = 16384

NUM_CORES = 2
NUM_SUBCORES = 16
NQ = 4
B_PER_Q = BATCH // NQ
LANES = 16

_mesh = plsc.VectorSubcoreMesh(core_axis_name="c", subcore_axis_name="s")


@functools.partial(
    pl.kernel,
    mesh=_mesh,
    out_type=jax.ShapeDtypeStruct((EMBED_DIM, BATCH), jnp.float32),
    scratch_types=[
        pltpu.VMEM((B_PER_Q,), jnp.int32),
        pltpu.VMEM((1, POSE_NUM), jnp.float32),
        pltpu.VMEM((1, B_PER_Q), jnp.float32),
    ],
    compiler_params=pltpu.CompilerParams(
        needs_layout_passes=False,
        skip_device_barrier=True,
        disable_bounds_checks=True,
        disable_semaphore_checks=True,
    ),
)
def _sc_gather(idx_hbm, tt_hbm, out_hbm, idx_v, row_v, outp_v):
    c = lax.axis_index("c")
    s = lax.axis_index("s")
    j = lax.rem(s, EMBED_DIM)
    q = c * 2 + lax.div(s, EMBED_DIM)

    @pl.when(s < 12)
    def _():
        pltpu.sync_copy(idx_hbm.at[pl.ds(q * B_PER_Q, B_PER_Q)], idx_v)
        pltpu.sync_copy(tt_hbm.at[pl.ds(j, 1)], row_v)
        zvec = jnp.zeros((LANES,), jnp.int32)
        for g in range(B_PER_Q // LANES):
            ivec = idx_v[pl.ds(g * LANES, LANES)]
            outp_v[0, pl.ds(g * LANES, LANES)] = plsc.load_gather(
                row_v, [zvec, ivec]
            )
        pltpu.sync_copy(
            outp_v, out_hbm.at[pl.ds(j, 1), pl.ds(q * B_PER_Q, B_PER_Q)]
        )


def kernel(indices, table):
    outt = _sc_gather(indices.astype(jnp.int32), table.T)
    return outt.T


# speedup vs baseline: 1.0216x; 1.0216x over previous
import functools

import jax
import jax.numpy as jnp
from jax import lax
from jax.experimental import pallas as pl
from jax.experimental.pallas import tpu as pltpu
from jax.experimental.pallas import tpu_sc as plsc

POSE_NUM = 100000
EMBED_DIM = 6
BATCH = 16384

NUM_CORES = 2
NQ = 4
B_PER_Q = BATCH // NQ
LANES = 16

_mesh = plsc.VectorSubcoreMesh(core_axis_name="c", subcore_axis_name="s")


@functools.partial(
    pl.kernel,
    mesh=_mesh,
    out_type=jax.ShapeDtypeStruct((EMBED_DIM, BATCH), jnp.float32),
    scratch_types=[
        pltpu.VMEM((B_PER_Q,), jnp.int32),
        pltpu.VMEM((1, POSE_NUM), jnp.float32),
        pltpu.VMEM((1, B_PER_Q), jnp.float32),
        pltpu.SemaphoreType.DMA,
    ],
    compiler_params=pltpu.CompilerParams(needs_layout_passes=False),
)
def _sc_gather(idx_hbm, tt_hbm, out_hbm, idx_v, row_v, outp_v, sem):
    c = lax.axis_index("c")
    s = lax.axis_index("s")
    j = lax.rem(s, EMBED_DIM)
    q = c * 2 + lax.div(s, EMBED_DIM)

    @pl.when(s < 12)
    def _():
        idx_cp = pltpu.async_copy(
            idx_hbm.at[pl.ds(q * B_PER_Q, B_PER_Q)], idx_v, sem
        )
        row_cp = pltpu.async_copy(tt_hbm.at[pl.ds(j, 1)], row_v, sem)
        idx_cp.wait()
        row_cp.wait()
        zvec = jnp.zeros((LANES,), jnp.int32)
        for g in range(B_PER_Q // LANES):
            ivec = idx_v[pl.ds(g * LANES, LANES)]
            outp_v[0, pl.ds(g * LANES, LANES)] = plsc.load_gather(
                row_v, [zvec, ivec]
            )
        pltpu.sync_copy(
            outp_v, out_hbm.at[pl.ds(j, 1), pl.ds(q * B_PER_Q, B_PER_Q)]
        )


def kernel(indices, table):
    outt = _sc_gather(indices.astype(jnp.int32), table.T)
    return outt.T
